# baseline (speedup 1.0000x reference)
import functools
import math

import jax
import jax.numpy as jnp
import numpy as np
from jax import lax
from jax.experimental import pallas as pl
from jax.experimental.pallas import tpu as pltpu
from jax.experimental.pallas import tpu_sc as plsc

H, W = 512, 1024
FP = 64
NC, NS = 2, 16
NW = NC * NS


def _row_kernel_tables(h):
    thetas = np.linspace(0.5, h - 0.5, num=h) * math.pi / h
    weight = 1.0 / np.sin(thetas)
    residual = weight % 2
    mask = residual >= 1
    residual[mask] -= 1
    residual[~mask] += 1
    n_ones = (weight - residual).astype(np.int32)
    edge_values = ((weight - n_ones) / 2.0).astype(np.float32)
    lengths = (n_ones + 2).astype(np.int32)
    return lengths, edge_values


_L_np, _E_np = _row_kernel_tables(H)


def _weights_kernel(yg_ref, lt_ref, et_ref, wf_ref, idx_ref):
    b = pl.program_id(0)
    f = yg_ref.shape[2]
    hp = jnp.float32

    xs_row = jnp.rint(yg_ref[0, 0:1, :] * (W - 1))
    ys_row = jnp.rint(yg_ref[0, 1:2, :] * (H - 1))

    jj = lax.broadcasted_iota(jnp.int32, (f, f), 0)
    kk = lax.broadcasted_iota(jnp.int32, (f, f), 1)
    ident = (jj == kk).astype(hp)
    xs_col = jnp.sum(ident * xs_row, axis=1, keepdims=True)
    ys_col = jnp.sum(ident * ys_row, axis=1, keepdims=True)
    xs_ci = xs_col.astype(jnp.int32)
    ys_ci = ys_col.astype(jnp.int32)

    iota_h_col = lax.broadcasted_iota(jnp.int32, (f, H), 1)
    e_sel = (ys_ci == iota_h_col).astype(hp)
    len_col = jnp.sum(e_sel * lt_ref[:, :], axis=1, keepdims=True)
    edge_col = jnp.sum(e_sel * et_ref[:, :], axis=1, keepdims=True)
    len_ci = len_col.astype(jnp.int32)

    pole = (ys_ci == 0) | (ys_ci == H - 1)
    len_ci = jnp.where(pole, W, len_ci)
    edge_col = jnp.where(pole, 1.0, edge_col)

    left = jnp.where(pole, 0, xs_ci - len_ci // 2)
    pw = lax.broadcasted_iota(jnp.int32, (f, W), 1)
    d = lax.rem(pw - left + 2 * W, W)
    cov = (d < len_ci).astype(hp)
    vals = jnp.where((d == 0) | (d == len_ci - 1), edge_col, 1.0)

    later_same = ((kk > jj) & (ys_ci == ys_row)).astype(jnp.bfloat16)
    later_cov = lax.dot_general(
        later_same, cov.astype(jnp.bfloat16), (((1,), (0,)), ((), ())),
        preferred_element_type=hp)
    weights = cov * (later_cov < 0.5).astype(hp) * vals

    wf_ref[0, :, :] = jnp.zeros((FP, W), dtype=hp)
    wf_ref[0, 0:f, :] = weights

    lane = lax.broadcasted_iota(jnp.int32, (1, FP), 1)
    ys_pad = jnp.concatenate(
        [ys_row.astype(jnp.int32),
         jnp.zeros((1, FP - f), dtype=jnp.int32)], axis=1)
    idx_ref[0, 0:1, :] = b * H + jnp.where(lane < f, ys_pad, 0)


def _sc_gather_dot_kernel(ypf_hbm, wf_hbm, idx_hbm, out_hbm,
                          idx_v, rows_v, w_v, acc_v, sem):
    wid = lax.axis_index("s") * NC + lax.axis_index("c")
    rows = rows_v.shape[0]
    base = wid * rows

    pltpu.sync_copy(idx_hbm.at[pl.ds(base, rows)], idx_v)
    gather = pltpu.async_copy(ypf_hbm.at[idx_v], rows_v, sem)
    pltpu.sync_copy(wf_hbm.at[pl.ds(base, rows)], w_v)
    gather.wait()

    nchunk = W // 16

    def body(i, accs):
        r = i // (nchunk // 4)
        cb = (i % (nchunk // 4)) * 4
        new = []
        for u in range(4):
            s = (cb + u) * 16
            prod = rows_v[r, pl.ds(s, 16)] * w_v[r, pl.ds(s, 16)]
            new.append(accs[u] + prod)
        return tuple(new)

    zero = jnp.zeros((16,), jnp.float32)
    accs = lax.fori_loop(0, rows * (nchunk // 4), body,
                         (zero, zero, zero, zero))
    acc_v[...] = (accs[0] + accs[1]) + (accs[2] + accs[3])
    pltpu.sync_copy(acc_v, out_hbm.at[wid])


def _combine_kernel(parts_ref, eps_ref, out_ref, *, nfix, nbatch):
    total = jnp.sum(parts_ref[:, :], dtype=jnp.float32)
    fc = jnp.full((1, 1), float(nfix), dtype=jnp.float32)
    eps_v = eps_ref[:, :]
    nf = jnp.where(fc < eps_v, eps_v, fc)
    out_ref[:, :] = jnp.reshape(total, (1, 1)) / (nf * float(nbatch))


def kernel(y_pred, y_gt, eps=1e-05):
    b, _, h, w = y_pred.shape
    f = y_gt.shape[1]
    yg = jnp.transpose(y_gt, (0, 2, 1))
    lt = jnp.asarray(_L_np, dtype=jnp.float32).reshape(1, h)
    et = jnp.asarray(_E_np, dtype=jnp.float32).reshape(1, h)
    eps_a = jnp.asarray(eps, dtype=jnp.float32).reshape(1, 1)

    wf, idx = pl.pallas_call(
        _weights_kernel,
        grid=(b,),
        in_specs=[
            pl.BlockSpec((1, 2, f), lambda i: (i, 0, 0)),
            pl.BlockSpec((1, h), lambda i: (0, 0)),
            pl.BlockSpec((1, h), lambda i: (0, 0)),
        ],
        out_specs=[
            pl.BlockSpec((1, FP, w), lambda i: (i, 0, 0)),
            pl.BlockSpec((1, 1, FP), lambda i: (i, 0, 0)),
        ],
        out_shape=[
            jax.ShapeDtypeStruct((b, FP, w), jnp.float32),
            jax.ShapeDtypeStruct((b, 1, FP), jnp.int32),
        ],
    )(yg, lt, et)

    ypf = y_pred.reshape(b * h, w)
    wf2 = wf.reshape(b * FP, w)
    idxf = idx.reshape(b * FP)
    rows_per_w = (b * FP) // NW

    mesh = plsc.VectorSubcoreMesh(
        core_axis_name="c", subcore_axis_name="s",
        num_cores=NC, num_subcores=NS)
    sc_fn = pl.kernel(
        _sc_gather_dot_kernel,
        out_type=jax.ShapeDtypeStruct((NW, 16), jnp.float32),
        mesh=mesh,
        scratch_types=[
            pltpu.VMEM((rows_per_w,), jnp.int32),
            pltpu.VMEM((rows_per_w, w), jnp.float32),
            pltpu.VMEM((rows_per_w, w), jnp.float32),
            pltpu.VMEM((16,), jnp.float32),
            pltpu.SemaphoreType.DMA,
        ],
    )
    parts = sc_fn(ypf, wf2, idxf)

    out = pl.pallas_call(
        functools.partial(_combine_kernel, nfix=f, nbatch=b),
        in_specs=[
            pl.BlockSpec((NW, 16), lambda: (0, 0)),
            pl.BlockSpec((1, 1), lambda: (0, 0)),
        ],
        out_specs=pl.BlockSpec((1, 1), lambda: (0, 0)),
        out_shape=jax.ShapeDtypeStruct((1, 1), jnp.float32),
    )(parts, eps_a)
    return jnp.reshape(out, ())


# speedup vs baseline: 1.0218x; 1.0218x over previous
import functools
import math

import jax
import jax.numpy as jnp
import numpy as np
from jax import lax
from jax.experimental import pallas as pl
from jax.experimental.pallas import tpu as pltpu
from jax.experimental.pallas import tpu_sc as plsc

H, W = 512, 1024
FP = 64
NC, NS = 2, 16
NW = NC * NS


def _row_kernel_tables(h):
    thetas = np.linspace(0.5, h - 0.5, num=h) * math.pi / h
    weight = 1.0 / np.sin(thetas)
    residual = weight % 2
    mask = residual >= 1
    residual[mask] -= 1
    residual[~mask] += 1
    n_ones = (weight - residual).astype(np.int32)
    edge_values = ((weight - n_ones) / 2.0).astype(np.float32)
    lengths = (n_ones + 2).astype(np.int32)
    return lengths, edge_values


_L_np, _E_np = _row_kernel_tables(H)


def _index_kernel(yg_ref, idx_ref):
    nb = yg_ref.shape[0]
    f = yg_ref.shape[2]
    ys = jnp.rint(yg_ref[:, 1, :] * (H - 1)).astype(jnp.int32)
    ys_pad = jnp.concatenate(
        [ys, jnp.zeros((nb, FP - f), dtype=jnp.int32)], axis=1)
    lane = lax.broadcasted_iota(jnp.int32, (nb, FP), 1)
    base = lax.broadcasted_iota(jnp.int32, (nb, FP), 0) * H
    idx_ref[:, :] = base + jnp.where(lane < f, ys_pad, 0)


def _sc_gather_kernel(ypf_hbm, idx_hbm, out_hbm, idx_v, rows_v, sem):
    wid = lax.axis_index("s") * NC + lax.axis_index("c")
    rows = rows_v.shape[0]
    base = wid * rows
    pltpu.sync_copy(idx_hbm.at[pl.ds(base, rows)], idx_v)
    pltpu.async_copy(ypf_hbm.at[idx_v], rows_v, sem).wait()
    pltpu.sync_copy(rows_v, out_hbm.at[pl.ds(base, rows)])


def _weights_dot_kernel(yg_ref, lt_ref, et_ref, g_ref, eps_ref, out_ref):
    b = pl.program_id(0)
    nb = pl.num_programs(0)
    f = yg_ref.shape[2]
    hp = jnp.float32

    xs_row = jnp.rint(yg_ref[0, 0:1, :] * (W - 1))
    ys_row = jnp.rint(yg_ref[0, 1:2, :] * (H - 1))

    jj = lax.broadcasted_iota(jnp.int32, (f, f), 0)
    kk = lax.broadcasted_iota(jnp.int32, (f, f), 1)
    ident = (jj == kk).astype(hp)
    xs_col = jnp.sum(ident * xs_row, axis=1, keepdims=True)
    ys_col = jnp.sum(ident * ys_row, axis=1, keepdims=True)
    xs_ci = xs_col.astype(jnp.int32)
    ys_ci = ys_col.astype(jnp.int32)

    iota_h_col = lax.broadcasted_iota(jnp.int32, (f, H), 1)
    e_sel = (ys_ci == iota_h_col).astype(hp)
    len_col = jnp.sum(e_sel * lt_ref[:, :], axis=1, keepdims=True)
    edge_col = jnp.sum(e_sel * et_ref[:, :], axis=1, keepdims=True)
    len_ci = len_col.astype(jnp.int32)

    pole = (ys_ci == 0) | (ys_ci == H - 1)
    len_ci = jnp.where(pole, W, len_ci)
    edge_col = jnp.where(pole, 1.0, edge_col)

    left = jnp.where(pole, 0, xs_ci - len_ci // 2)
    pw = lax.broadcasted_iota(jnp.int32, (f, W), 1)
    d = lax.rem(pw - left + 2 * W, W)
    cov = (d < len_ci).astype(hp)
    vals = jnp.where((d == 0) | (d == len_ci - 1), edge_col, 1.0)

    later_same = ((kk > jj) & (ys_ci == ys_row)).astype(jnp.bfloat16)
    later_cov = lax.dot_general(
        later_same, cov.astype(jnp.bfloat16), (((1,), (0,)), ((), ())),
        preferred_element_type=hp)
    weights = cov * (later_cov < 0.5).astype(hp) * vals

    s = jnp.sum(weights * g_ref[0, 0:f, :], dtype=hp)

    @pl.when(b == 0)
    def _():
        out_ref[:, :] = jnp.zeros_like(out_ref)

    out_ref[:, :] += jnp.reshape(s, (1, 1))

    @pl.when(b == nb - 1)
    def _():
        fc = jnp.full((1, 1), float(f), dtype=hp)
        eps_v = eps_ref[:, :]
        nf = jnp.where(fc < eps_v, eps_v, fc)
        out_ref[:, :] = out_ref[:, :] / (nf * float(nb))


def kernel(y_pred, y_gt, eps=1e-05):
    b, _, h, w = y_pred.shape
    f = y_gt.shape[1]
    yg = jnp.transpose(y_gt, (0, 2, 1))
    lt = jnp.asarray(_L_np, dtype=jnp.float32).reshape(1, h)
    et = jnp.asarray(_E_np, dtype=jnp.float32).reshape(1, h)
    eps_a = jnp.asarray(eps, dtype=jnp.float32).reshape(1, 1)

    idx = pl.pallas_call(
        _index_kernel,
        in_specs=[pl.BlockSpec((b, 2, f), lambda: (0, 0, 0))],
        out_specs=pl.BlockSpec((b, FP), lambda: (0, 0)),
        out_shape=jax.ShapeDtypeStruct((b, FP), jnp.int32),
    )(yg)

    ypf = y_pred.reshape(b * h, w)
    idxf = idx.reshape(b * FP)
    rows_per_w = (b * FP) // NW

    mesh = plsc.VectorSubcoreMesh(
        core_axis_name="c", subcore_axis_name="s",
        num_cores=NC, num_subcores=NS)
    sc_fn = pl.kernel(
        _sc_gather_kernel,
        out_type=jax.ShapeDtypeStruct((b * FP, w), jnp.float32),
        mesh=mesh,
        scratch_types=[
            pltpu.VMEM((rows_per_w,), jnp.int32),
            pltpu.VMEM((rows_per_w, w), jnp.float32),
            pltpu.SemaphoreType.DMA,
        ],
    )
    g = sc_fn(ypf, idxf)
    g3 = g.reshape(b, FP, w)

    out = pl.pallas_call(
        _weights_dot_kernel,
        grid=(b,),
        in_specs=[
            pl.BlockSpec((1, 2, f), lambda i: (i, 0, 0)),
            pl.BlockSpec((1, h), lambda i: (0, 0)),
            pl.BlockSpec((1, h), lambda i: (0, 0)),
            pl.BlockSpec((1, FP, w), lambda i: (i, 0, 0)),
            pl.BlockSpec((1, 1), lambda i: (0, 0)),
        ],
        out_specs=pl.BlockSpec((1, 1), lambda i: (0, 0)),
        out_shape=jax.ShapeDtypeStruct((1, 1), jnp.float32),
    )(yg, lt, et, g3, eps_a)
    return jnp.reshape(out, ())


# speedup vs baseline: 2.3558x; 2.3055x over previous
import math

import jax
import jax.numpy as jnp
import numpy as np
from jax.experimental import pallas as pl
from jax.experimental.pallas import tpu as pltpu

H, W = 512, 1024


def _row_kernel_tables(h):
    thetas = np.linspace(0.5, h - 0.5, num=h) * math.pi / h
    weight = 1.0 / np.sin(thetas)
    residual = weight % 2
    mask = residual >= 1
    residual[mask] -= 1
    residual[~mask] += 1
    n_ones = (weight - residual).astype(np.int32)
    edge_values = ((weight - n_ones) / 2.0).astype(np.float32)
    lengths = (n_ones + 2).astype(np.int32)
    return lengths, edge_values


_L_np, _E_np = _row_kernel_tables(H)


def _sample_contrib(yg_ref, yp_ref, lt_ref, et_ref, i):
    f = yg_ref.shape[2]
    hp = jnp.float32

    xs_row = jnp.rint(yg_ref[i, 0:1, :] * (W - 1))
    ys_row = jnp.rint(yg_ref[i, 1:2, :] * (H - 1))

    jj = jax.lax.broadcasted_iota(jnp.int32, (f, f), 0)
    kk = jax.lax.broadcasted_iota(jnp.int32, (f, f), 1)
    ident = (jj == kk).astype(hp)
    xs_col = jnp.sum(ident * xs_row, axis=1, keepdims=True)
    ys_col = jnp.sum(ident * ys_row, axis=1, keepdims=True)
    xs_ci = xs_col.astype(jnp.int32)
    ys_ci = ys_col.astype(jnp.int32)

    iota_h_col = jax.lax.broadcasted_iota(jnp.int32, (f, H), 1)
    e_sel = (ys_ci == iota_h_col).astype(hp)

    len_col = jnp.sum(e_sel * lt_ref[:, :], axis=1, keepdims=True)
    edge_col = jnp.sum(e_sel * et_ref[:, :], axis=1, keepdims=True)
    len_ci = len_col.astype(jnp.int32)

    pole = (ys_ci == 0) | (ys_ci == H - 1)
    len_ci = jnp.where(pole, W, len_ci)
    edge_col = jnp.where(pole, 1.0, edge_col)

    left = jnp.where(pole, 0, xs_ci - len_ci // 2)
    pw = jax.lax.broadcasted_iota(jnp.int32, (f, W), 1)
    d = jax.lax.rem(pw - left + 2 * W, W)
    cov = (d < len_ci).astype(hp)
    vals = jnp.where((d == 0) | (d == len_ci - 1), edge_col, 1.0)

    later_same = ((kk > jj) & (ys_ci == ys_row)).astype(jnp.bfloat16)
    later_cov = jax.lax.dot_general(
        later_same, cov.astype(jnp.bfloat16), (((1,), (0,)), ((), ())),
        preferred_element_type=hp)
    weights = cov * (later_cov < 0.5).astype(hp) * vals

    yp = yp_ref[i]
    yp_hi = yp.astype(jnp.bfloat16)
    yp_lo = (yp - yp_hi.astype(hp)).astype(jnp.bfloat16)
    e16 = e_sel.astype(jnp.bfloat16)
    g = (jax.lax.dot_general(e16, yp_hi, (((1,), (0,)), ((), ())),
                             preferred_element_type=hp)
         + jax.lax.dot_general(e16, yp_lo, (((1,), (0,)), ((), ())),
                               preferred_element_type=hp))

    return jnp.sum(weights * g, dtype=hp)


def _fixation_loss_kernel(yg_ref, yp_ref, lt_ref, et_ref, eps_ref, out_ref):
    b = pl.program_id(0)
    nb = pl.num_programs(0)
    f = yg_ref.shape[2]
    spb = yg_ref.shape[0]
    hp = jnp.float32

    s = jnp.float32(0.0)
    for i in range(spb):
        s = s + _sample_contrib(yg_ref, yp_ref, lt_ref, et_ref, i)

    @pl.when(b == 0)
    def _():
        out_ref[:, :] = jnp.zeros_like(out_ref)

    out_ref[:, :] += jnp.reshape(s, (1, 1))

    @pl.when(b == nb - 1)
    def _():
        fc = jnp.full((1, 1), float(f), dtype=hp)
        eps_v = eps_ref[:, :]
        nf = jnp.where(fc < eps_v, eps_v, fc)
        out_ref[:, :] = out_ref[:, :] / (nf * float(nb * spb))


def kernel(y_pred, y_gt, eps=1e-05):
    b, _, h, w = y_pred.shape
    f = y_gt.shape[1]
    yp = y_pred.reshape(b, h, w)
    yg = jnp.transpose(y_gt, (0, 2, 1))
    lt = jnp.asarray(_L_np, dtype=jnp.float32).reshape(1, h)
    et = jnp.asarray(_E_np, dtype=jnp.float32).reshape(1, h)
    eps_a = jnp.asarray(eps, dtype=jnp.float32).reshape(1, 1)

    spb = 4
    out = pl.pallas_call(
        _fixation_loss_kernel,
        grid=(b // spb,),
        in_specs=[
            pl.BlockSpec((spb, 2, f), lambda i: (i, 0, 0)),
            pl.BlockSpec((spb, h, w), lambda i: (i, 0, 0)),
            pl.BlockSpec((1, h), lambda i: (0, 0)),
            pl.BlockSpec((1, h), lambda i: (0, 0)),
            pl.BlockSpec((1, 1), lambda i: (0, 0)),
        ],
        out_specs=pl.BlockSpec((1, 1), lambda i: (0, 0)),
        out_shape=jax.ShapeDtypeStruct((1, 1), jnp.float32),
    )(yg, yp, lt, et, eps_a)
    return jnp.reshape(out, ())
